# Initial kernel scaffold; baseline (speedup 1.0000x reference)
import jax
import jax.numpy as jnp
from jax.experimental import pallas as pl


def kernel(input_ids, position_ids, pos_table):
    raise NotImplementedError("write your pallas kernel here")


# speedup vs baseline: 1.0747x; 1.0747x over previous
import functools

import jax
import jax.numpy as jnp
from jax import lax
from jax.experimental import pallas as pl
from jax.experimental.pallas import tpu as pltpu
from jax.experimental.pallas import tpu_sc as plsc

B, S, D = 4, 8192, 1024
N = B * S
NC, NS = 2, 16
NW = NC * NS
ROWS_PER_W = N // NW
CHUNK = 32
N_CHUNKS = ROWS_PER_W // CHUNK
VPR = D // 16


def _body(x_hbm, ids_hbm, table_hbm, out_hbm, idx_v, ibuf, gbuf, sem_i, sem_g):
    wid = lax.axis_index("s") * NC + lax.axis_index("c")
    pltpu.sync_copy(ids_hbm.at[wid], idx_v)

    def chunk_step(c, _):
        base = wid * ROWS_PER_W + c * CHUNK
        cp_i = pltpu.make_async_copy(x_hbm.at[pl.ds(base, CHUNK)], ibuf, sem_i)
        cp_i.start()
        cp_g = pltpu.make_async_copy(table_hbm.at[idx_v.at[c]], gbuf, sem_g)
        cp_g.start()
        cp_i.wait()
        cp_g.wait()

        def row_add(r, _):
            for j in range(VPR):
                sl = pl.ds(j * 16, 16)
                gbuf[r, sl] = gbuf[r, sl] + ibuf[r, sl]
            return _

        lax.fori_loop(0, CHUNK, row_add, None)
        pltpu.sync_copy(gbuf, out_hbm.at[pl.ds(base, CHUNK)])
        return _

    lax.fori_loop(0, N_CHUNKS, chunk_step, None)


@jax.jit
def _pos_embed(x, ids, table):
    mesh = plsc.VectorSubcoreMesh(
        core_axis_name="c", subcore_axis_name="s", num_cores=NC, num_subcores=NS
    )
    return pl.kernel(
        _body,
        out_type=jax.ShapeDtypeStruct((N, D), jnp.float32),
        mesh=mesh,
        scratch_types=[
            pltpu.VMEM((N_CHUNKS, CHUNK), jnp.int32),
            pltpu.VMEM((CHUNK, D), jnp.float32),
            pltpu.VMEM((CHUNK, D), jnp.float32),
            pltpu.SemaphoreType.DMA,
            pltpu.SemaphoreType.DMA,
        ],
    )(x, ids, table)


def kernel(input_ids, position_ids, pos_table):
    x = input_ids.reshape(N, D)
    ids = position_ids.astype(jnp.int32).reshape(NW, N_CHUNKS, CHUNK)
    out = _pos_embed(x, ids, pos_table)
    return out.reshape(B, S, D)


# speedup vs baseline: 1.8261x; 1.6992x over previous
import jax
import jax.numpy as jnp
from jax import lax
from jax.experimental import pallas as pl
from jax.experimental.pallas import tpu as pltpu
from jax.experimental.pallas import tpu_sc as plsc

B, S, D = 4, 8192, 1024
N = B * S
NC, NS = 2, 16
NW = NC * NS
ROWS_PER_W = N // NW
CHUNK = 8
NCH = ROWS_PER_W // CHUNK
NB = 4
VPR = D // 16


def _body(x_hbm, ids_hbm, table_hbm, out_hbm, idx_v, ibuf, gbuf,
          sems_i, sems_g, sems_o):
    wid = lax.axis_index("s") * NC + lax.axis_index("c")
    row0 = wid * ROWS_PER_W
    pltpu.sync_copy(ids_hbm.at[wid], idx_v)

    def in_copy(c, b):
        return pltpu.make_async_copy(
            x_hbm.at[pl.ds(row0 + c * CHUNK, CHUNK)], ibuf.at[b], sems_i[b])

    def g_copy(c, b):
        return pltpu.make_async_copy(
            table_hbm.at[idx_v.at[c]], gbuf.at[b], sems_g[b])

    def out_copy(c, b):
        return pltpu.make_async_copy(
            gbuf.at[b], out_hbm.at[pl.ds(row0 + c * CHUNK, CHUNK)], sems_o[b])

    def issue(c, b):
        in_copy(c, b).start()
        g_copy(c, b).start()

    def compute(b):
        def row_add(r, _):
            for j in range(VPR):
                sl = pl.ds(j * 16, 16)
                gbuf[b, r, sl] = gbuf[b, r, sl] + ibuf[b, r, sl]
            return _

        lax.fori_loop(0, CHUNK, row_add, None)

    def sub_step(c, b, la_wait):
        in_copy(c, b).wait()
        g_copy(c, b).wait()
        la, bq = c + 2, (b + 2) % NB
        if la_wait:
            out_copy(0, bq).wait()
        issue(la, bq)
        compute(b)
        out_copy(c, b).start()

    issue(0, 0)
    issue(1, 1)

    sub_step(0, 0, False)
    sub_step(1, 1, False)
    sub_step(2, 2, True)
    sub_step(3, 3, True)

    def group(g, _):
        c0 = g * NB
        for b in range(NB):
            sub_step(c0 + b, b, True)
        return _

    lax.fori_loop(1, NCH // NB - 1, group, None)

    for c in range(NCH - NB, NCH):
        b = c % NB
        in_copy(c, b).wait()
        g_copy(c, b).wait()
        if c + 2 < NCH:
            out_copy(0, (b + 2) % NB).wait()
            issue(c + 2, (b + 2) % NB)
        compute(b)
        out_copy(c, b).start()

    for b in range(NB):
        out_copy(0, b).wait()


@jax.jit
def _pos_embed(x, ids, table):
    mesh = plsc.VectorSubcoreMesh(
        core_axis_name="c", subcore_axis_name="s", num_cores=NC, num_subcores=NS
    )
    return pl.kernel(
        _body,
        out_type=jax.ShapeDtypeStruct((N, D), jnp.float32),
        mesh=mesh,
        scratch_types=[
            pltpu.VMEM((NCH, CHUNK), jnp.int32),
            pltpu.VMEM((NB, CHUNK, D), jnp.float32),
            pltpu.VMEM((NB, CHUNK, D), jnp.float32),
            [pltpu.SemaphoreType.DMA] * NB,
            [pltpu.SemaphoreType.DMA] * NB,
            [pltpu.SemaphoreType.DMA] * NB,
        ],
    )(x, ids, table)


def kernel(input_ids, position_ids, pos_table):
    x = input_ids.reshape(N, D)
    ids = position_ids.astype(jnp.int32).reshape(NW, NCH, CHUNK)
    out = _pos_embed(x, ids, pos_table)
    return out.reshape(B, S, D)
